# Initial kernel scaffold; baseline (speedup 1.0000x reference)
import jax
import jax.numpy as jnp
from jax.experimental import pallas as pl


def kernel(x, batch, pos, W1, b1, W2, b2, scaling, gumbel, active_mask):
    raise NotImplementedError("write your pallas kernel here")


# speedup vs baseline: 18.6477x; 18.6477x over previous
import functools

import jax
import jax.numpy as jnp
from jax.experimental import pallas as pl
from jax.experimental.pallas import tpu as pltpu

N_TOK = 8192
N_FEAT = 128
N_SUPER = 32
N_BATCH = 8
BLK = 512
NBLK = N_TOK // BLK
_HI = jax.lax.Precision.HIGHEST
_DEF = jax.lax.Precision.DEFAULT


def _fused_kernel(x_ref, b_ref, pos_ref, w1_ref, b1_ref, w2_ref, b2_ref,
                  sc_ref, g_ref, am_ref,
                  s_out, out_ref, mu_ref, aux_ref,
                  denom_ref, muT_ref, ent_ref):
    i = pl.program_id(0)

    @pl.when(i == 0)
    def _init():
        out_ref[...] = jnp.zeros_like(out_ref)
        mu_ref[...] = jnp.zeros_like(mu_ref)
        denom_ref[...] = jnp.zeros_like(denom_ref)
        muT_ref[...] = jnp.zeros_like(muT_ref)
        aux_ref[...] = jnp.zeros_like(aux_ref)
        ent_ref[0, 0] = 0.0

    x = x_ref[...]
    h = jnp.maximum(
        jax.lax.dot(x, w1_ref[...], precision=_DEF) + b1_ref[...], 0.0)
    logits = (jax.lax.dot(h, w2_ref[...], precision=_DEF)
              + b2_ref[...]) * sc_ref[0, 0]
    logits = jnp.where(am_ref[...] == 0.0, -1e9, logits)
    z = logits + g_ref[...]
    m = jnp.max(z, axis=-1, keepdims=True)
    e = jnp.exp(z - m)
    s = e / jnp.sum(e, axis=-1, keepdims=True)
    s_out[...] = s

    ent_ref[0, 0] += jnp.sum(s * jnp.log(s + 1e-9))

    bcol = b_ref[...]
    pos = pos_ref[...]
    for b in range(N_BATCH):
        mask = (bcol == b).astype(jnp.float32)
        sb = s * mask
        denom_ref[b:b + 1, :] += jnp.sum(sb, axis=0, keepdims=True)
        numb = jax.lax.dot_general(sb, x, (((0,), (0,)), ((), ())),
                                   precision=_HI)
        out_ref[b, :, :] += numb
        posb = jax.lax.dot_general(sb, pos, (((0,), (0,)), ((), ())),
                                   precision=_HI)
        mu_ref[b, :, :] += posb
        posTb = jax.lax.dot_general(pos, sb, (((0,), (0,)), ((), ())),
                                    precision=_HI)
        muT_ref[b, :, :] += posTb

    @pl.when(i == NBLK - 1)
    def _epilogue():
        K = N_SUPER
        denom = denom_ref[...]
        dsafe = denom + 1e-9
        out_ref[...] = out_ref[...] / dsafe[:, :, None]
        mu_ref[...] = mu_ref[...] / dsafe[:, :, None]

        avg2 = jnp.sum(denom, axis=0, keepdims=True) / N_TOK
        entropy = -ent_ref[0, 0] / N_TOK
        u = 1.0 / K
        diversity = jnp.sum(u * (jnp.log(u) - jnp.log(avg2 + 1e-9)))
        am = am_ref[...]
        pruning = jnp.mean(jnp.abs(avg2 * (1.0 - am)))
        sparsity = jnp.sum(am) / K
        collapse = jnp.maximum(jnp.max(avg2) - u, 0.0)
        balance = jnp.sum((avg2 - u) ** 2) * K

        eyec = 1.0 - jnp.eye(K, dtype=jnp.float32)
        sep = 0.0
        for b in range(N_BATCH):
            dinv = 1.0 / (denom_ref[b:b + 1, :] + 1e-9)
            mx = mu_ref[b, :, 0:1]
            my = mu_ref[b, :, 1:2]
            mxT = muT_ref[b, 0:1, :] * dinv
            myT = muT_ref[b, 1:2, :] * dinv
            dx = mx - mxT
            dy = my - myT
            d = dx * dx + dy * dy
            sep += jnp.sum(jnp.exp(-d) * eyec)
        separation = sep / (N_BATCH * K * K)

        aux_ref[0:1, 0:1] = jnp.reshape(entropy, (1, 1))
        aux_ref[0:1, 1:2] = jnp.reshape(diversity, (1, 1))
        aux_ref[0:1, 2:3] = jnp.reshape(pruning, (1, 1))
        aux_ref[0:1, 3:4] = jnp.reshape(sparsity, (1, 1))
        aux_ref[0:1, 4:5] = jnp.reshape(collapse, (1, 1))
        aux_ref[0:1, 5:6] = jnp.reshape(balance, (1, 1))
        aux_ref[0:1, 6:7] = jnp.reshape(separation, (1, 1))


@jax.jit
def _run(x, batch2d, pos, W1, b1r, W2, b2r, scaling2d, gumbel, am2d):
    grid = (NBLK,)
    out_shapes = [
        jax.ShapeDtypeStruct((N_TOK, N_SUPER), jnp.float32),
        jax.ShapeDtypeStruct((N_BATCH, N_SUPER, N_FEAT), jnp.float32),
        jax.ShapeDtypeStruct((N_BATCH, N_SUPER, 2), jnp.float32),
        jax.ShapeDtypeStruct((8, 128), jnp.float32),
    ]
    in_specs = [
        pl.BlockSpec((BLK, N_FEAT), lambda i: (i, 0)),
        pl.BlockSpec((BLK, 1), lambda i: (i, 0)),
        pl.BlockSpec((BLK, 2), lambda i: (i, 0)),
        pl.BlockSpec((N_FEAT, N_FEAT), lambda i: (0, 0)),
        pl.BlockSpec((1, N_FEAT), lambda i: (0, 0)),
        pl.BlockSpec((N_FEAT, N_SUPER), lambda i: (0, 0)),
        pl.BlockSpec((1, N_SUPER), lambda i: (0, 0)),
        pl.BlockSpec(memory_space=pltpu.SMEM),
        pl.BlockSpec((BLK, N_SUPER), lambda i: (i, 0)),
        pl.BlockSpec((1, N_SUPER), lambda i: (0, 0)),
    ]
    out_specs = [
        pl.BlockSpec((BLK, N_SUPER), lambda i: (i, 0)),
        pl.BlockSpec((N_BATCH, N_SUPER, N_FEAT), lambda i: (0, 0, 0)),
        pl.BlockSpec((N_BATCH, N_SUPER, 2), lambda i: (0, 0, 0)),
        pl.BlockSpec((8, 128), lambda i: (0, 0)),
    ]
    scratch = [
        pltpu.VMEM((N_BATCH, N_SUPER), jnp.float32),
        pltpu.VMEM((N_BATCH, 2, N_SUPER), jnp.float32),
        pltpu.SMEM((1, 1), jnp.float32),
    ]
    return pl.pallas_call(
        _fused_kernel,
        grid=grid,
        in_specs=in_specs,
        out_specs=out_specs,
        out_shape=out_shapes,
        scratch_shapes=scratch,
        compiler_params=pltpu.CompilerParams(
            dimension_semantics=("arbitrary",)),
    )(x, batch2d, pos, W1, b1r, W2, b2r, scaling2d, gumbel, am2d)


def kernel(x, batch, pos, W1, b1, W2, b2, scaling, gumbel, active_mask):
    batch2d = batch.reshape(N_TOK, 1)
    s, out, mu, aux = _run(
        x, batch2d, pos, W1, b1.reshape(1, N_FEAT), W2,
        b2.reshape(1, N_SUPER), scaling.reshape(1, 1), gumbel,
        active_mask.reshape(1, N_SUPER))
    entropy = aux[0, 0]
    diversity = aux[0, 1]
    pruning = aux[0, 2]
    sparsity = aux[0, 3]
    collapse = aux[0, 4]
    balance = aux[0, 5]
    separation = aux[0, 6]
    zero = jnp.zeros((), jnp.float32)
    return (out, s, mu, entropy, diversity, zero, pruning, sparsity,
            zero, collapse, balance, separation)


# speedup vs baseline: 29.0951x; 1.5602x over previous
import functools

import jax
import jax.numpy as jnp
from jax.experimental import pallas as pl
from jax.experimental.pallas import tpu as pltpu

N_TOK = 8192
N_FEAT = 128
N_SUPER = 32
N_BATCH = 8
BLK = 512
NBLK = N_TOK // BLK
_HI = jax.lax.Precision.HIGHEST
_DEF = jax.lax.Precision.DEFAULT


def _fused_kernel(x_ref, b_ref, pos_ref, w1_ref, b1_ref, w2_ref, b2_ref,
                  sc_ref, g_ref, am_ref,
                  s_out, out_ref, mu_ref, aux_ref,
                  denom_ref, muT_ref, ent_ref):
    i = pl.program_id(0)

    @pl.when(i == 0)
    def _init():
        out_ref[...] = jnp.zeros_like(out_ref)
        mu_ref[...] = jnp.zeros_like(mu_ref)
        denom_ref[...] = jnp.zeros_like(denom_ref)
        muT_ref[...] = jnp.zeros_like(muT_ref)
        aux_ref[...] = jnp.zeros_like(aux_ref)
        ent_ref[0, 0] = 0.0

    x = x_ref[...]
    h = jnp.maximum(
        jax.lax.dot(x, w1_ref[...], precision=_DEF) + b1_ref[...], 0.0)
    logits = (jax.lax.dot(h, w2_ref[...], precision=_DEF)
              + b2_ref[...]) * sc_ref[0, 0]
    logits = jnp.where(am_ref[...] == 0.0, -1e9, logits)
    z = logits + g_ref[...]
    m = jnp.max(z, axis=-1, keepdims=True)
    e = jnp.exp(z - m)
    s = e / jnp.sum(e, axis=-1, keepdims=True)
    s_out[...] = s

    ent_ref[0, 0] += jnp.sum(s * jnp.log(s + 1e-9))

    bcol = b_ref[...]
    pos = pos_ref[...]
    blo = b_ref[0, 0]
    bhi = b_ref[BLK - 1, 0]
    for b in range(N_BATCH):
        @pl.when((b >= blo) & (b <= bhi))
        def _pool():
            mask = (bcol == b).astype(jnp.float32)
            sb = s * mask
            denom_ref[b:b + 1, :] += jnp.sum(sb, axis=0, keepdims=True)
            numb = jax.lax.dot_general(sb, x, (((0,), (0,)), ((), ())),
                                       precision=_HI)
            out_ref[b, :, :] += numb
            posb = jax.lax.dot_general(sb, pos, (((0,), (0,)), ((), ())),
                                       precision=_HI)
            mu_ref[b, :, :] += posb
            posTb = jax.lax.dot_general(pos, sb, (((0,), (0,)), ((), ())),
                                        precision=_HI)
            muT_ref[b, :, :] += posTb

    @pl.when(i == NBLK - 1)
    def _epilogue():
        K = N_SUPER
        denom = denom_ref[...]
        dsafe = denom + 1e-9
        out_ref[...] = out_ref[...] / dsafe[:, :, None]
        mu_ref[...] = mu_ref[...] / dsafe[:, :, None]

        avg2 = jnp.sum(denom, axis=0, keepdims=True) / N_TOK
        entropy = -ent_ref[0, 0] / N_TOK
        u = 1.0 / K
        diversity = jnp.sum(u * (jnp.log(u) - jnp.log(avg2 + 1e-9)))
        am = am_ref[...]
        pruning = jnp.mean(jnp.abs(avg2 * (1.0 - am)))
        sparsity = jnp.sum(am) / K
        collapse = jnp.maximum(jnp.max(avg2) - u, 0.0)
        balance = jnp.sum((avg2 - u) ** 2) * K

        eyec = 1.0 - jnp.eye(K, dtype=jnp.float32)
        sep = 0.0
        for b in range(N_BATCH):
            dinv = 1.0 / (denom_ref[b:b + 1, :] + 1e-9)
            mx = mu_ref[b, :, 0:1]
            my = mu_ref[b, :, 1:2]
            mxT = muT_ref[b, 0:1, :] * dinv
            myT = muT_ref[b, 1:2, :] * dinv
            dx = mx - mxT
            dy = my - myT
            d = dx * dx + dy * dy
            sep += jnp.sum(jnp.exp(-d) * eyec)
        separation = sep / (N_BATCH * K * K)

        aux_ref[0:1, 0:1] = jnp.reshape(entropy, (1, 1))
        aux_ref[0:1, 1:2] = jnp.reshape(diversity, (1, 1))
        aux_ref[0:1, 2:3] = jnp.reshape(pruning, (1, 1))
        aux_ref[0:1, 3:4] = jnp.reshape(sparsity, (1, 1))
        aux_ref[0:1, 4:5] = jnp.reshape(collapse, (1, 1))
        aux_ref[0:1, 5:6] = jnp.reshape(balance, (1, 1))
        aux_ref[0:1, 6:7] = jnp.reshape(separation, (1, 1))


@jax.jit
def _run(x, batch2d, pos, W1, b1r, W2, b2r, scaling2d, gumbel, am2d):
    grid = (NBLK,)
    out_shapes = [
        jax.ShapeDtypeStruct((N_TOK, N_SUPER), jnp.float32),
        jax.ShapeDtypeStruct((N_BATCH, N_SUPER, N_FEAT), jnp.float32),
        jax.ShapeDtypeStruct((N_BATCH, N_SUPER, 2), jnp.float32),
        jax.ShapeDtypeStruct((8, 128), jnp.float32),
    ]
    in_specs = [
        pl.BlockSpec((BLK, N_FEAT), lambda i: (i, 0)),
        pl.BlockSpec((BLK, 1), lambda i: (i, 0)),
        pl.BlockSpec((BLK, 2), lambda i: (i, 0)),
        pl.BlockSpec((N_FEAT, N_FEAT), lambda i: (0, 0)),
        pl.BlockSpec((1, N_FEAT), lambda i: (0, 0)),
        pl.BlockSpec((N_FEAT, N_SUPER), lambda i: (0, 0)),
        pl.BlockSpec((1, N_SUPER), lambda i: (0, 0)),
        pl.BlockSpec(memory_space=pltpu.SMEM),
        pl.BlockSpec((BLK, N_SUPER), lambda i: (i, 0)),
        pl.BlockSpec((1, N_SUPER), lambda i: (0, 0)),
    ]
    out_specs = [
        pl.BlockSpec((BLK, N_SUPER), lambda i: (i, 0)),
        pl.BlockSpec((N_BATCH, N_SUPER, N_FEAT), lambda i: (0, 0, 0)),
        pl.BlockSpec((N_BATCH, N_SUPER, 2), lambda i: (0, 0, 0)),
        pl.BlockSpec((8, 128), lambda i: (0, 0)),
    ]
    scratch = [
        pltpu.VMEM((N_BATCH, N_SUPER), jnp.float32),
        pltpu.VMEM((N_BATCH, 2, N_SUPER), jnp.float32),
        pltpu.SMEM((1, 1), jnp.float32),
    ]
    return pl.pallas_call(
        _fused_kernel,
        grid=grid,
        in_specs=in_specs,
        out_specs=out_specs,
        out_shape=out_shapes,
        scratch_shapes=scratch,
        compiler_params=pltpu.CompilerParams(
            dimension_semantics=("arbitrary",)),
    )(x, batch2d, pos, W1, b1r, W2, b2r, scaling2d, gumbel, am2d)


def kernel(x, batch, pos, W1, b1, W2, b2, scaling, gumbel, active_mask):
    batch2d = batch.reshape(N_TOK, 1)
    s, out, mu, aux = _run(
        x, batch2d, pos, W1, b1.reshape(1, N_FEAT), W2,
        b2.reshape(1, N_SUPER), scaling.reshape(1, 1), gumbel,
        active_mask.reshape(1, N_SUPER))
    entropy = aux[0, 0]
    diversity = aux[0, 1]
    pruning = aux[0, 2]
    sparsity = aux[0, 3]
    collapse = aux[0, 4]
    balance = aux[0, 5]
    separation = aux[0, 6]
    zero = jnp.zeros((), jnp.float32)
    return (out, s, mu, entropy, diversity, zero, pruning, sparsity,
            zero, collapse, balance, separation)


# speedup vs baseline: 32.0231x; 1.1006x over previous
import jax
import jax.numpy as jnp
from jax.experimental import pallas as pl
from jax.experimental.pallas import tpu as pltpu

N_TOK = 8192
N_FEAT = 128
N_SUPER = 32
N_BATCH = 8
BLK = 512
NBLK = N_TOK // BLK
_HI = jax.lax.Precision.HIGHEST
_DEF = jax.lax.Precision.DEFAULT


def _fused_kernel(x_ref, b_ref, pos_ref, w1_ref, b1_ref, w2_ref, b2_ref,
                  sc_ref, g_ref, am_ref,
                  s_out, out_ref, muT_ref, aux_ref,
                  denom_ref, ent_ref):
    i = pl.program_id(0)

    @pl.when(i == 0)
    def _init():
        out_ref[...] = jnp.zeros_like(out_ref)
        muT_ref[...] = jnp.zeros_like(muT_ref)
        denom_ref[...] = jnp.zeros_like(denom_ref)
        aux_ref[...] = jnp.zeros_like(aux_ref)
        ent_ref[0, 0] = 0.0

    x = x_ref[...]
    h = jnp.maximum(
        jax.lax.dot(x, w1_ref[...], precision=_DEF) + b1_ref[...], 0.0)
    logits = (jax.lax.dot(h, w2_ref[...], precision=_DEF)
              + b2_ref[...]) * sc_ref[0, 0]
    logits = jnp.where(am_ref[...] == 0.0, -1e9, logits)
    z = logits + g_ref[...]
    m = jnp.max(z, axis=-1, keepdims=True)
    e = jnp.exp(z - m)
    s = e / jnp.sum(e, axis=-1, keepdims=True)
    s_out[...] = s

    ent_ref[0, 0] += jnp.sum(s * jnp.log(s + 1e-9))

    bcol = b_ref[...]
    posx = pos_ref[:, 0:1]
    posy = pos_ref[:, 1:2]
    blo = b_ref[0, 0]
    bhi = b_ref[BLK - 1, 0]
    for b in range(N_BATCH):
        @pl.when((b >= blo) & (b <= bhi))
        def _pool():
            mask = (bcol == b).astype(jnp.float32)
            sb = s * mask
            denom_ref[b:b + 1, :] += jnp.sum(sb, axis=0, keepdims=True)
            numb = jax.lax.dot_general(sb, x, (((0,), (0,)), ((), ())),
                                       precision=_HI)
            out_ref[b, :, :] += numb
            muT_ref[b, 0:1, :] += jnp.sum(sb * posx, axis=0, keepdims=True)
            muT_ref[b, 1:2, :] += jnp.sum(sb * posy, axis=0, keepdims=True)

    @pl.when(i == NBLK - 1)
    def _epilogue():
        K = N_SUPER
        denom = denom_ref[...]
        dsafe = denom + 1e-9
        out_ref[...] = out_ref[...] / dsafe[:, :, None]
        muT_ref[...] = muT_ref[...] / dsafe[:, None, :]

        avg2 = jnp.sum(denom, axis=0, keepdims=True) / N_TOK
        entropy = -ent_ref[0, 0] / N_TOK
        u = 1.0 / K
        diversity = jnp.sum(u * (jnp.log(u) - jnp.log(avg2 + 1e-9)))
        am = am_ref[...]
        pruning = jnp.mean(jnp.abs(avg2 * (1.0 - am)))
        sparsity = jnp.sum(am) / K
        collapse = jnp.maximum(jnp.max(avg2) - u, 0.0)
        balance = jnp.sum((avg2 - u) ** 2) * K

        eyec = 1.0 - jnp.eye(K, dtype=jnp.float32)
        ones21 = jnp.ones((2, 1), dtype=jnp.float32)
        sep = 0.0
        for b in range(N_BATCH):
            mub2 = muT_ref[b]
            sq = mub2 * mub2
            n2row = jnp.sum(sq, axis=0, keepdims=True)
            n2col = jax.lax.dot_general(sq, ones21,
                                        (((0,), (0,)), ((), ())),
                                        precision=_HI)
            G = jax.lax.dot_general(mub2, mub2,
                                    (((0,), (0,)), ((), ())),
                                    precision=_HI)
            d = n2col + n2row - 2.0 * G
            sep += jnp.sum(jnp.exp(-d) * eyec)
        separation = sep / (N_BATCH * K * K)

        aux_ref[0:1, 0:1] = jnp.reshape(entropy, (1, 1))
        aux_ref[0:1, 1:2] = jnp.reshape(diversity, (1, 1))
        aux_ref[0:1, 2:3] = jnp.reshape(pruning, (1, 1))
        aux_ref[0:1, 3:4] = jnp.reshape(sparsity, (1, 1))
        aux_ref[0:1, 4:5] = jnp.reshape(collapse, (1, 1))
        aux_ref[0:1, 5:6] = jnp.reshape(balance, (1, 1))
        aux_ref[0:1, 6:7] = jnp.reshape(separation, (1, 1))


@jax.jit
def _run(x, batch2d, pos, W1, b1r, W2, b2r, scaling2d, gumbel, am2d):
    grid = (NBLK,)
    out_shapes = [
        jax.ShapeDtypeStruct((N_TOK, N_SUPER), jnp.float32),
        jax.ShapeDtypeStruct((N_BATCH, N_SUPER, N_FEAT), jnp.float32),
        jax.ShapeDtypeStruct((N_BATCH, 2, N_SUPER), jnp.float32),
        jax.ShapeDtypeStruct((8, 128), jnp.float32),
    ]
    in_specs = [
        pl.BlockSpec((BLK, N_FEAT), lambda i: (i, 0)),
        pl.BlockSpec((BLK, 1), lambda i: (i, 0)),
        pl.BlockSpec((BLK, 2), lambda i: (i, 0)),
        pl.BlockSpec((N_FEAT, N_FEAT), lambda i: (0, 0)),
        pl.BlockSpec((1, N_FEAT), lambda i: (0, 0)),
        pl.BlockSpec((N_FEAT, N_SUPER), lambda i: (0, 0)),
        pl.BlockSpec((1, N_SUPER), lambda i: (0, 0)),
        pl.BlockSpec(memory_space=pltpu.SMEM),
        pl.BlockSpec((BLK, N_SUPER), lambda i: (i, 0)),
        pl.BlockSpec((1, N_SUPER), lambda i: (0, 0)),
    ]
    out_specs = [
        pl.BlockSpec((BLK, N_SUPER), lambda i: (i, 0)),
        pl.BlockSpec((N_BATCH, N_SUPER, N_FEAT), lambda i: (0, 0, 0)),
        pl.BlockSpec((N_BATCH, 2, N_SUPER), lambda i: (0, 0, 0)),
        pl.BlockSpec((8, 128), lambda i: (0, 0)),
    ]
    scratch = [
        pltpu.VMEM((N_BATCH, N_SUPER), jnp.float32),
        pltpu.SMEM((1, 1), jnp.float32),
    ]
    return pl.pallas_call(
        _fused_kernel,
        grid=grid,
        in_specs=in_specs,
        out_specs=out_specs,
        out_shape=out_shapes,
        scratch_shapes=scratch,
        compiler_params=pltpu.CompilerParams(
            dimension_semantics=("arbitrary",)),
    )(x, batch2d, pos, W1, b1r, W2, b2r, scaling2d, gumbel, am2d)


def kernel(x, batch, pos, W1, b1, W2, b2, scaling, gumbel, active_mask):
    batch2d = batch.reshape(N_TOK, 1)
    s, out, muT, aux = _run(
        x, batch2d, pos, W1, b1.reshape(1, N_FEAT), W2,
        b2.reshape(1, N_SUPER), scaling.reshape(1, 1), gumbel,
        active_mask.reshape(1, N_SUPER))
    mu = jnp.swapaxes(muT, 1, 2)
    entropy = aux[0, 0]
    diversity = aux[0, 1]
    pruning = aux[0, 2]
    sparsity = aux[0, 3]
    collapse = aux[0, 4]
    balance = aux[0, 5]
    separation = aux[0, 6]
    zero = jnp.zeros((), jnp.float32)
    return (out, s, mu, entropy, diversity, zero, pruning, sparsity,
            zero, collapse, balance, separation)


# speedup vs baseline: 32.5737x; 1.0172x over previous
import jax
import jax.numpy as jnp
from jax.experimental import pallas as pl
from jax.experimental.pallas import tpu as pltpu

N_TOK = 8192
N_FEAT = 128
N_SUPER = 32
N_BATCH = 8
BLK = 512
NBLK = N_TOK // BLK
_HI = jax.lax.Precision.HIGHEST
_DEF = jax.lax.Precision.DEFAULT


def _fused_kernel(x_ref, b_ref, pos_ref, w1_ref, b1_ref, w2_ref, b2_ref,
                  sc_ref, g_ref, am_ref,
                  s_out, out_ref, muT_ref, aux_ref,
                  denom_ref, ent_ref):
    i = pl.program_id(0)

    @pl.when(i == 0)
    def _init():
        out_ref[...] = jnp.zeros_like(out_ref)
        muT_ref[...] = jnp.zeros_like(muT_ref)
        denom_ref[...] = jnp.zeros_like(denom_ref)
        aux_ref[...] = jnp.zeros_like(aux_ref)
        ent_ref[0, 0] = 0.0

    x = x_ref[...]
    h = jnp.maximum(
        jax.lax.dot(x, w1_ref[...], precision=_DEF) + b1_ref[...], 0.0)
    logits = (jax.lax.dot(h, w2_ref[...], precision=_DEF)
              + b2_ref[...]) * sc_ref[0, 0]
    logits = jnp.where(am_ref[...] == 0.0, -1e9, logits)
    z = logits + g_ref[...]
    m = jnp.max(z, axis=-1, keepdims=True)
    e = jnp.exp(z - m)
    s = e / jnp.sum(e, axis=-1, keepdims=True)
    s_out[...] = s

    ent_ref[0, 0] += jnp.sum(s * jnp.log(s + 1e-9))

    bcol = b_ref[...]
    pos = pos_ref[...]
    aug = jnp.concatenate(
        [jnp.ones((BLK, 1), jnp.float32), pos], axis=1)
    blo = b_ref[0, 0]
    bhi = b_ref[BLK - 1, 0]

    def _accum_rows(sb):
        numb = jax.lax.dot_general(sb, x, (((0,), (0,)), ((), ())),
                                   precision=_DEF)
        auxr = jax.lax.dot_general(aug, sb, (((0,), (0,)), ((), ())),
                                   precision=_HI)
        return numb, auxr

    @pl.when(blo == bhi)
    def _single():
        numb, auxr = _accum_rows(s)
        bs = pl.ds(blo, 1)
        out_ref[bs, :, :] += numb.reshape(1, N_SUPER, N_FEAT)
        denom_ref[bs, :] += auxr[0:1, :]
        muT_ref[bs, :, :] += auxr[1:3, :].reshape(1, 2, N_SUPER)

    @pl.when(blo != bhi)
    def _multi():
        for b in range(N_BATCH):
            @pl.when((b >= blo) & (b <= bhi))
            def _pool():
                mask = (bcol == b).astype(jnp.float32)
                numb, auxr = _accum_rows(s * mask)
                out_ref[b, :, :] += numb
                denom_ref[b:b + 1, :] += auxr[0:1, :]
                muT_ref[b, :, :] += auxr[1:3, :]

    @pl.when(i == NBLK - 1)
    def _epilogue():
        K = N_SUPER
        denom = denom_ref[...]
        dsafe = denom + 1e-9
        out_ref[...] = out_ref[...] / dsafe[:, :, None]
        muT_ref[...] = muT_ref[...] / dsafe[:, None, :]

        avg2 = jnp.sum(denom, axis=0, keepdims=True) / N_TOK
        entropy = -ent_ref[0, 0] / N_TOK
        u = 1.0 / K
        diversity = jnp.sum(u * (jnp.log(u) - jnp.log(avg2 + 1e-9)))
        am = am_ref[...]
        pruning = jnp.mean(jnp.abs(avg2 * (1.0 - am)))
        sparsity = jnp.sum(am) / K
        collapse = jnp.maximum(jnp.max(avg2) - u, 0.0)
        balance = jnp.sum((avg2 - u) ** 2) * K

        eyec = 1.0 - jnp.eye(K, dtype=jnp.float32)
        ones21 = jnp.ones((2, 1), dtype=jnp.float32)
        sep = 0.0
        for b in range(N_BATCH):
            mub2 = muT_ref[b]
            sq = mub2 * mub2
            n2row = jnp.sum(sq, axis=0, keepdims=True)
            n2col = jax.lax.dot_general(sq, ones21,
                                        (((0,), (0,)), ((), ())),
                                        precision=_HI)
            G = jax.lax.dot_general(mub2, mub2,
                                    (((0,), (0,)), ((), ())),
                                    precision=_HI)
            d = n2col + n2row - 2.0 * G
            sep += jnp.sum(jnp.exp(-d) * eyec)
        separation = sep / (N_BATCH * K * K)

        aux_ref[0:1, 0:1] = jnp.reshape(entropy, (1, 1))
        aux_ref[0:1, 1:2] = jnp.reshape(diversity, (1, 1))
        aux_ref[0:1, 2:3] = jnp.reshape(pruning, (1, 1))
        aux_ref[0:1, 3:4] = jnp.reshape(sparsity, (1, 1))
        aux_ref[0:1, 4:5] = jnp.reshape(collapse, (1, 1))
        aux_ref[0:1, 5:6] = jnp.reshape(balance, (1, 1))
        aux_ref[0:1, 6:7] = jnp.reshape(separation, (1, 1))


@jax.jit
def _run(x, batch2d, pos, W1, b1r, W2, b2r, scaling2d, gumbel, am2d):
    grid = (NBLK,)
    out_shapes = [
        jax.ShapeDtypeStruct((N_TOK, N_SUPER), jnp.float32),
        jax.ShapeDtypeStruct((N_BATCH, N_SUPER, N_FEAT), jnp.float32),
        jax.ShapeDtypeStruct((N_BATCH, 2, N_SUPER), jnp.float32),
        jax.ShapeDtypeStruct((8, 128), jnp.float32),
    ]
    in_specs = [
        pl.BlockSpec((BLK, N_FEAT), lambda i: (i, 0)),
        pl.BlockSpec((BLK, 1), lambda i: (i, 0)),
        pl.BlockSpec((BLK, 2), lambda i: (i, 0)),
        pl.BlockSpec((N_FEAT, N_FEAT), lambda i: (0, 0)),
        pl.BlockSpec((1, N_FEAT), lambda i: (0, 0)),
        pl.BlockSpec((N_FEAT, N_SUPER), lambda i: (0, 0)),
        pl.BlockSpec((1, N_SUPER), lambda i: (0, 0)),
        pl.BlockSpec(memory_space=pltpu.SMEM),
        pl.BlockSpec((BLK, N_SUPER), lambda i: (i, 0)),
        pl.BlockSpec((1, N_SUPER), lambda i: (0, 0)),
    ]
    out_specs = [
        pl.BlockSpec((BLK, N_SUPER), lambda i: (i, 0)),
        pl.BlockSpec((N_BATCH, N_SUPER, N_FEAT), lambda i: (0, 0, 0)),
        pl.BlockSpec((N_BATCH, 2, N_SUPER), lambda i: (0, 0, 0)),
        pl.BlockSpec((8, 128), lambda i: (0, 0)),
    ]
    scratch = [
        pltpu.VMEM((N_BATCH, N_SUPER), jnp.float32),
        pltpu.SMEM((1, 1), jnp.float32),
    ]
    return pl.pallas_call(
        _fused_kernel,
        grid=grid,
        in_specs=in_specs,
        out_specs=out_specs,
        out_shape=out_shapes,
        scratch_shapes=scratch,
        compiler_params=pltpu.CompilerParams(
            dimension_semantics=("arbitrary",)),
    )(x, batch2d, pos, W1, b1r, W2, b2r, scaling2d, gumbel, am2d)


def kernel(x, batch, pos, W1, b1, W2, b2, scaling, gumbel, active_mask):
    batch2d = batch.reshape(N_TOK, 1)
    s, out, muT, aux = _run(
        x, batch2d, pos, W1, b1.reshape(1, N_FEAT), W2,
        b2.reshape(1, N_SUPER), scaling.reshape(1, 1), gumbel,
        active_mask.reshape(1, N_SUPER))
    mu = jnp.swapaxes(muT, 1, 2)
    entropy = aux[0, 0]
    diversity = aux[0, 1]
    pruning = aux[0, 2]
    sparsity = aux[0, 3]
    collapse = aux[0, 4]
    balance = aux[0, 5]
    separation = aux[0, 6]
    zero = jnp.zeros((), jnp.float32)
    return (out, s, mu, entropy, diversity, zero, pruning, sparsity,
            zero, collapse, balance, separation)


# speedup vs baseline: 36.9311x; 1.1338x over previous
import jax
import jax.numpy as jnp
from jax.experimental import pallas as pl
from jax.experimental.pallas import tpu as pltpu

N_TOK = 8192
N_FEAT = 128
N_SUPER = 32
N_BATCH = 8
BLK = 1024
NBLK = N_TOK // BLK
_HI = jax.lax.Precision.HIGHEST
_DEF = jax.lax.Precision.DEFAULT


def _fused_kernel(x_ref, b_ref, pos_ref, w1_ref, b1_ref, w2_ref, b2_ref,
                  sc_ref, g_ref, am_ref,
                  s_out, out_ref, muT_ref, aux_ref,
                  denom_ref, ent_ref):
    i = pl.program_id(0)

    @pl.when(i == 0)
    def _init():
        out_ref[...] = jnp.zeros_like(out_ref)
        muT_ref[...] = jnp.zeros_like(muT_ref)
        denom_ref[...] = jnp.zeros_like(denom_ref)
        aux_ref[...] = jnp.zeros_like(aux_ref)
        ent_ref[0, 0] = 0.0

    x = x_ref[...]
    h = jnp.maximum(
        jax.lax.dot(x, w1_ref[...], precision=_DEF) + b1_ref[...], 0.0)
    logits = (jax.lax.dot(h, w2_ref[...], precision=_DEF)
              + b2_ref[...]) * sc_ref[0, 0]
    logits = jnp.where(am_ref[...] == 0.0, -1e9, logits)
    z = logits + g_ref[...]
    m = jnp.max(z, axis=-1, keepdims=True)
    e = jnp.exp(z - m)
    s = e / jnp.sum(e, axis=-1, keepdims=True)
    s_out[...] = s

    ent_ref[0, 0] += jnp.sum(s * jnp.log(s + 1e-9))

    bcol = b_ref[...]
    pos = pos_ref[...]
    aug = jnp.concatenate(
        [jnp.ones((BLK, 1), jnp.float32), pos], axis=1)
    blo = b_ref[0, 0]
    bhi = b_ref[BLK - 1, 0]

    def _accum_rows(sb):
        numb = jax.lax.dot_general(sb, x, (((0,), (0,)), ((), ())),
                                   precision=_DEF)
        auxr = jax.lax.dot_general(aug, sb, (((0,), (0,)), ((), ())),
                                   precision=_HI)
        return numb, auxr

    @pl.when(blo == bhi)
    def _single():
        numb, auxr = _accum_rows(s)
        bs = pl.ds(blo, 1)
        out_ref[bs, :, :] += numb.reshape(1, N_SUPER, N_FEAT)
        denom_ref[bs, :] += auxr[0:1, :]
        muT_ref[bs, :, :] += auxr[1:3, :].reshape(1, 2, N_SUPER)

    @pl.when(blo != bhi)
    def _multi():
        for b in range(N_BATCH):
            @pl.when((b >= blo) & (b <= bhi))
            def _pool():
                mask = (bcol == b).astype(jnp.float32)
                numb, auxr = _accum_rows(s * mask)
                out_ref[b, :, :] += numb
                denom_ref[b:b + 1, :] += auxr[0:1, :]
                muT_ref[b, :, :] += auxr[1:3, :]

    @pl.when(i == NBLK - 1)
    def _epilogue():
        K = N_SUPER
        denom = denom_ref[...]
        dsafe = denom + 1e-9
        out_ref[...] = out_ref[...] / dsafe[:, :, None]
        muT_ref[...] = muT_ref[...] / dsafe[:, None, :]

        avg2 = jnp.sum(denom, axis=0, keepdims=True) / N_TOK
        entropy = -ent_ref[0, 0] / N_TOK
        u = 1.0 / K
        diversity = jnp.sum(u * (jnp.log(u) - jnp.log(avg2 + 1e-9)))
        am = am_ref[...]
        pruning = jnp.mean(jnp.abs(avg2 * (1.0 - am)))
        sparsity = jnp.sum(am) / K
        collapse = jnp.maximum(jnp.max(avg2) - u, 0.0)
        balance = jnp.sum((avg2 - u) ** 2) * K

        eyec = 1.0 - jnp.eye(K, dtype=jnp.float32)
        ones21 = jnp.ones((2, 1), dtype=jnp.float32)
        sep = 0.0
        for b in range(N_BATCH):
            mub2 = muT_ref[b]
            sq = mub2 * mub2
            n2row = jnp.sum(sq, axis=0, keepdims=True)
            n2col = jax.lax.dot_general(sq, ones21,
                                        (((0,), (0,)), ((), ())),
                                        precision=_HI)
            G = jax.lax.dot_general(mub2, mub2,
                                    (((0,), (0,)), ((), ())),
                                    precision=_HI)
            d = n2col + n2row - 2.0 * G
            sep += jnp.sum(jnp.exp(-d) * eyec)
        separation = sep / (N_BATCH * K * K)

        aux_ref[0:1, 0:1] = jnp.reshape(entropy, (1, 1))
        aux_ref[0:1, 1:2] = jnp.reshape(diversity, (1, 1))
        aux_ref[0:1, 2:3] = jnp.reshape(pruning, (1, 1))
        aux_ref[0:1, 3:4] = jnp.reshape(sparsity, (1, 1))
        aux_ref[0:1, 4:5] = jnp.reshape(collapse, (1, 1))
        aux_ref[0:1, 5:6] = jnp.reshape(balance, (1, 1))
        aux_ref[0:1, 6:7] = jnp.reshape(separation, (1, 1))


@jax.jit
def _run(x, batch2d, pos, W1, b1r, W2, b2r, scaling2d, gumbel, am2d):
    grid = (NBLK,)
    out_shapes = [
        jax.ShapeDtypeStruct((N_TOK, N_SUPER), jnp.float32),
        jax.ShapeDtypeStruct((N_BATCH, N_SUPER, N_FEAT), jnp.float32),
        jax.ShapeDtypeStruct((N_BATCH, 2, N_SUPER), jnp.float32),
        jax.ShapeDtypeStruct((8, 128), jnp.float32),
    ]
    in_specs = [
        pl.BlockSpec((BLK, N_FEAT), lambda i: (i, 0)),
        pl.BlockSpec((BLK, 1), lambda i: (i, 0)),
        pl.BlockSpec((BLK, 2), lambda i: (i, 0)),
        pl.BlockSpec((N_FEAT, N_FEAT), lambda i: (0, 0)),
        pl.BlockSpec((1, N_FEAT), lambda i: (0, 0)),
        pl.BlockSpec((N_FEAT, N_SUPER), lambda i: (0, 0)),
        pl.BlockSpec((1, N_SUPER), lambda i: (0, 0)),
        pl.BlockSpec(memory_space=pltpu.SMEM),
        pl.BlockSpec((BLK, N_SUPER), lambda i: (i, 0)),
        pl.BlockSpec((1, N_SUPER), lambda i: (0, 0)),
    ]
    out_specs = [
        pl.BlockSpec((BLK, N_SUPER), lambda i: (i, 0)),
        pl.BlockSpec((N_BATCH, N_SUPER, N_FEAT), lambda i: (0, 0, 0)),
        pl.BlockSpec((N_BATCH, 2, N_SUPER), lambda i: (0, 0, 0)),
        pl.BlockSpec((8, 128), lambda i: (0, 0)),
    ]
    scratch = [
        pltpu.VMEM((N_BATCH, N_SUPER), jnp.float32),
        pltpu.SMEM((1, 1), jnp.float32),
    ]
    return pl.pallas_call(
        _fused_kernel,
        grid=grid,
        in_specs=in_specs,
        out_specs=out_specs,
        out_shape=out_shapes,
        scratch_shapes=scratch,
        compiler_params=pltpu.CompilerParams(
            dimension_semantics=("arbitrary",)),
    )(x, batch2d, pos, W1, b1r, W2, b2r, scaling2d, gumbel, am2d)


def kernel(x, batch, pos, W1, b1, W2, b2, scaling, gumbel, active_mask):
    batch2d = batch.reshape(N_TOK, 1)
    s, out, muT, aux = _run(
        x, batch2d, pos, W1, b1.reshape(1, N_FEAT), W2,
        b2.reshape(1, N_SUPER), scaling.reshape(1, 1), gumbel,
        active_mask.reshape(1, N_SUPER))
    mu = jnp.swapaxes(muT, 1, 2)
    entropy = aux[0, 0]
    diversity = aux[0, 1]
    pruning = aux[0, 2]
    sparsity = aux[0, 3]
    collapse = aux[0, 4]
    balance = aux[0, 5]
    separation = aux[0, 6]
    zero = jnp.zeros((), jnp.float32)
    return (out, s, mu, entropy, diversity, zero, pruning, sparsity,
            zero, collapse, balance, separation)
